# Initial kernel scaffold; baseline (speedup 1.0000x reference)
import jax
import jax.numpy as jnp
from jax.experimental import pallas as pl


def kernel(x, edge_index, weights, W_in, W_out, latp, Ws1, b1, Ws2, b2, attn_l, attn_r, s_attn, beta, aw, theta):
    raise NotImplementedError("write your pallas kernel here")


# speedup vs baseline: 2.5682x; 2.5682x over previous
import functools

import jax
import jax.numpy as jnp
from jax import lax
from jax.experimental import pallas as pl
from jax.experimental.pallas import tpu as pltpu
from jax.experimental.pallas import tpu_sc as plsc

N = 10000
E = 320000
NFEAT = 128
NHID = 256
NC = 16
D = 128
ALPHA = 0.1
NEG_SLOPE = 0.2

NCORE = 2
NSUB = 16
NW = NCORE * NSUB
BPW = E // NW
CH = 400

HALF = N // 2
M = 5200

BN = 1000
BE = 3200


def _make_gather():
  nchunk = BPW // CH
  mesh = plsc.VectorSubcoreMesh(core_axis_name="c", subcore_axis_name="s")

  @functools.partial(
      pl.kernel, mesh=mesh,
      out_type=jax.ShapeDtypeStruct((E, D), jnp.float32),
      scratch_types=[
          pltpu.VMEM((CH,), jnp.int32),
          pltpu.VMEM((CH, D), jnp.float32),
          pltpu.SemaphoreType.DMA,
      ],
  )
  def gk(table_hbm, idx_hbm, out_hbm, idx_v, rows_v, sem):
    wid = lax.axis_index("s") * NCORE + lax.axis_index("c")
    base = wid * BPW
    for j in range(nchunk):
      off = base + j * CH
      pltpu.sync_copy(idx_hbm.at[pl.ds(off, CH)], idx_v)
      pltpu.async_copy(table_hbm.at[idx_v], rows_v, sem).wait()
      pltpu.sync_copy(rows_v, out_hbm.at[pl.ds(off, CH)])

  return gk


def _make_scatter_add():
  nchunk = BPW // CH
  zch = 400
  nzch = M // zch
  mesh = plsc.VectorSubcoreMesh(core_axis_name="c", subcore_axis_name="s")

  @functools.partial(
      pl.kernel, mesh=mesh,
      out_type=jax.ShapeDtypeStruct((2 * M, D), jnp.float32),
      scratch_types=[
          pltpu.VMEM((CH,), jnp.int32),
          pltpu.VMEM((CH, D), jnp.float32),
          pltpu.VMEM_SHARED((M, D), jnp.float32),
      ],
  )
  def sk(vals_hbm, idx_hbm, zeros_hbm, out_hbm, idx_v, vals_v, acc_sh):
    cid = lax.axis_index("c")
    sid = lax.axis_index("s")
    wid = sid * NCORE + cid

    @pl.when(sid == 0)
    def _():
      for r in range(nzch):
        pltpu.sync_copy(zeros_hbm.at[pl.ds(r * zch, zch)], vals_v)
        pltpu.sync_copy(vals_v, acc_sh.at[pl.ds(r * zch, zch)])

    plsc.subcore_barrier()
    base = wid * BPW
    for j in range(nchunk):
      off = base + j * CH
      pltpu.sync_copy(idx_hbm.at[pl.ds(off, CH)], idx_v)
      pltpu.sync_copy(vals_hbm.at[pl.ds(off, CH)], vals_v)
      pltpu.sync_copy(vals_v, acc_sh.at[idx_v], add=True)
    plsc.subcore_barrier()

    @pl.when(sid == 0)
    def _():
      for r in range(nzch):
        pltpu.sync_copy(acc_sh.at[pl.ds(r * zch, zch)], vals_v)
        pltpu.sync_copy(vals_v, out_hbm.at[pl.ds(cid * M + r * zch, zch)])

  return sk


_gatherD = _make_gather()
_scatterD = _make_scatter_add()


def _elu(v):
  return jnp.where(v > 0, v, jnp.exp(v) - 1.0)


def _stem_kern(x_ref, latp_ref, win_ref, wout_ref, ws1_ref, b1_ref, ws2_ref,
               b2_ref, al_ref, ar_ref, sa_ref, t_ref, f0_ref):
  h1 = _elu(jnp.dot(x_ref[...], win_ref[...],
                    preferred_element_type=jnp.float32))
  h = jnp.dot(h1, wout_ref[...], preferred_element_type=jnp.float32)
  hl1 = _elu(jnp.dot(latp_ref[...], ws1_ref[...],
                     preferred_element_type=jnp.float32) + b1_ref[...])
  hl = _elu(jnp.dot(hl1, ws2_ref[...],
                    preferred_element_type=jnp.float32) + b2_ref[...])
  lr = jnp.where(h > 0, h, NEG_SLOPE * h)
  el = jnp.sum(lr * al_ref[...], axis=1, keepdims=True)
  er = jnp.sum(lr * ar_ref[...], axis=1, keepdims=True)
  ones16 = jnp.ones((1, 16), dtype=jnp.float32)
  t_ref[:, 0:16] = h
  t_ref[:, 16:32] = hl
  t_ref[:, 32:48] = hl * sa_ref[...]
  t_ref[:, 48:64] = el * ones16
  t_ref[:, 64:80] = er * ones16
  t_ref[:, 80:128] = jnp.zeros((t_ref.shape[0], 48), jnp.float32)
  f0_ref[:, 0:16] = h
  f0_ref[:, 16:128] = jnp.zeros((f0_ref.shape[0], 112), jnp.float32)


def _stem(x, latp, W_in, W_out, Ws1, b1, Ws2, b2, attn_l, attn_r, s_attn):
  full = lambda i: (0, 0)
  return pl.pallas_call(
      _stem_kern,
      grid=(N // BN,),
      in_specs=[
          pl.BlockSpec((BN, NFEAT), lambda i: (i, 0)),
          pl.BlockSpec((BN, NC), lambda i: (i, 0)),
          pl.BlockSpec((NFEAT, NHID), full),
          pl.BlockSpec((NHID, NC), full),
          pl.BlockSpec((NC, NC), full),
          pl.BlockSpec((1, NC), full),
          pl.BlockSpec((NC, NC), full),
          pl.BlockSpec((1, NC), full),
          pl.BlockSpec((1, NC), full),
          pl.BlockSpec((1, NC), full),
          pl.BlockSpec((1, NC), full),
      ],
      out_specs=[
          pl.BlockSpec((BN, D), lambda i: (i, 0)),
          pl.BlockSpec((BN, D), lambda i: (i, 0)),
      ],
      out_shape=[
          jax.ShapeDtypeStruct((N, D), jnp.float32),
          jax.ShapeDtypeStruct((N, D), jnp.float32),
      ],
  )(x, latp, W_in, W_out, Ws1, b1, Ws2, b2, attn_l, attn_r, s_attn)


def _edge1_kern(gs_ref, gd_ref, w16_ref, wab_ref, betaw_ref, ew_ref, lp_ref):
  i = pl.program_id(0)
  gs = gs_ref[...]
  gd = gd_ref[...]
  fs = gs[:, 0:16]
  fd = gd[:, 0:16]
  hls = gs[:, 16:32]
  hld = gd[:, 16:32]
  sels = gs[:, 32:48]
  se = jnp.sum(sels * hld, axis=1, keepdims=True)
  e = gs[:, 48:49] + gd[:, 64:65] + se
  sdf = jnp.sum((fs - fd) ** 2, axis=1, keepdims=True)
  sds = jnp.sum((hls - hld) ** 2, axis=1, keepdims=True)
  d = wab_ref[0:1, 0:1] * sdf + wab_ref[0:1, 1:2] * sds
  ew = jnp.exp(e - betaw_ref[0:1, 0:1] * d) + 1e-9
  ew_ref[...] = jnp.broadcast_to(ew, ew_ref.shape)

  @pl.when(i == 0)
  def _():
    lp_ref[...] = jnp.zeros_like(lp_ref)

  lp_ref[...] += jnp.sum(hls * hld * w16_ref[...])[None, None]


def _edge1(gs, gd, w16, wab, betaw):
  full = lambda i: (0, 0)
  return pl.pallas_call(
      _edge1_kern,
      grid=(E // BE,),
      in_specs=[
          pl.BlockSpec((BE, D), lambda i: (i, 0)),
          pl.BlockSpec((BE, D), lambda i: (i, 0)),
          pl.BlockSpec((BE, NC), lambda i: (i, 0)),
          pl.BlockSpec((1, 2), full),
          pl.BlockSpec((1, 1), full),
      ],
      out_specs=[
          pl.BlockSpec((BE, D), lambda i: (i, 0)),
          pl.BlockSpec((1, 1), full),
      ],
      out_shape=[
          jax.ShapeDtypeStruct((E, D), jnp.float32),
          jax.ShapeDtypeStruct((1, 1), jnp.float32),
      ],
  )(gs, gd, w16, wab, betaw)


def _norm_kern(ow_ref, iw_ref, ns_ref, nd_ref):
  ow = ow_ref[0] + ow_ref[1]
  iw = iw_ref[0] + iw_ref[1]
  ns_ref[...] = lax.rsqrt(jnp.maximum(ow, 1e-12))
  nd_ref[...] = lax.rsqrt(jnp.maximum(iw, 1e-12))


def _norm(ow2, iw2):
  return pl.pallas_call(
      _norm_kern,
      grid=(HALF // BN,),
      in_specs=[
          pl.BlockSpec((2, BN, D), lambda i: (0, i, 0)),
          pl.BlockSpec((2, BN, D), lambda i: (0, i, 0)),
      ],
      out_specs=[
          pl.BlockSpec((BN, D), lambda i: (i, 0)),
          pl.BlockSpec((BN, D), lambda i: (i, 0)),
      ],
      out_shape=[
          jax.ShapeDtypeStruct((HALF, D), jnp.float32),
          jax.ShapeDtypeStruct((HALF, D), jnp.float32),
      ],
  )(ow2, iw2)


def _wmul_kern(ps_ref, pd_ref, ew_ref, perm_ref, w_ref):
  w_ref[...] = ps_ref[...] * pd_ref[...] * ew_ref[...] + perm_ref[0:1, 0:1]


def _wmul(ps, pd, ew, perm):
  full = lambda i: (0, 0)
  return pl.pallas_call(
      _wmul_kern,
      grid=(E // BE,),
      in_specs=[
          pl.BlockSpec((BE, D), lambda i: (i, 0)),
          pl.BlockSpec((BE, D), lambda i: (i, 0)),
          pl.BlockSpec((BE, D), lambda i: (i, 0)),
          pl.BlockSpec((1, 1), full),
      ],
      out_specs=pl.BlockSpec((BE, D), lambda i: (i, 0)),
      out_shape=jax.ShapeDtypeStruct((E, D), jnp.float32),
  )(ps, pd, ew, perm)


def _mul_kern(a_ref, b_ref, o_ref):
  o_ref[...] = a_ref[...] * b_ref[...]


def _mul(a, b):
  return pl.pallas_call(
      _mul_kern,
      grid=(E // BE,),
      in_specs=[
          pl.BlockSpec((BE, D), lambda i: (i, 0)),
          pl.BlockSpec((BE, D), lambda i: (i, 0)),
      ],
      out_specs=pl.BlockSpec((BE, D), lambda i: (i, 0)),
      out_shape=jax.ShapeDtypeStruct((E, D), jnp.float32),
  )(a, b)


def _axpy_kern(agg_ref, f0_ref, o_ref):
  agg = agg_ref[0] + agg_ref[1]
  o_ref[...] = (1.0 - ALPHA) * agg + ALPHA * f0_ref[...]


def _axpy(agg2, feat0):
  return pl.pallas_call(
      _axpy_kern,
      grid=(HALF // BN,),
      in_specs=[
          pl.BlockSpec((2, BN, D), lambda i: (0, i, 0)),
          pl.BlockSpec((BN, D), lambda i: (i, 0)),
      ],
      out_specs=pl.BlockSpec((BN, D), lambda i: (i, 0)),
      out_shape=jax.ShapeDtypeStruct((HALF, D), jnp.float32),
  )(agg2, feat0)


def _split_kern(v_ref, lo_ref, hi_ref):
  v = v_ref[...]
  lo = jnp.where(v < HALF, v, HALF)
  hi = jnp.where(v >= HALF, v - HALF, HALF)
  lo_ref[...] = lo
  hi_ref[...] = hi


def _split_idx(v2d):
  return pl.pallas_call(
      _split_kern,
      grid=(1,),
      in_specs=[pl.BlockSpec((2500, 128), lambda i: (0, 0))],
      out_specs=[
          pl.BlockSpec((2500, 128), lambda i: (0, 0)),
          pl.BlockSpec((2500, 128), lambda i: (0, 0)),
      ],
      out_shape=[
          jax.ShapeDtypeStruct((2500, 128), jnp.int32),
          jax.ShapeDtypeStruct((2500, 128), jnp.int32),
      ],
  )(v2d)


def _final_kern(f_ref, o_ref):
  y = _elu(f_ref[:, 0:16])
  m = jnp.max(y, axis=1, keepdims=True)
  z = y - m
  o_ref[...] = z - jnp.log(jnp.sum(jnp.exp(z), axis=1, keepdims=True))


def _final(feat):
  return pl.pallas_call(
      _final_kern,
      grid=(N // BN,),
      in_specs=[pl.BlockSpec((BN, D), lambda i: (i, 0))],
      out_specs=pl.BlockSpec((BN, NC), lambda i: (i, 0)),
      out_shape=jax.ShapeDtypeStruct((N, NC), jnp.float32),
  )(feat)


def kernel(x, edge_index, weights, W_in, W_out, latp, Ws1, b1, Ws2, b2,
           attn_l, attn_r, s_attn, beta, aw, theta):
  src = edge_index[0]
  dst = edge_index[1]
  b1r = b1.reshape(1, NC)
  b2r = b2.reshape(1, NC)

  wab = jax.nn.softmax(aw, axis=1)
  betaw = 2.0 / (jnp.exp(-beta) + 1.0)
  perm = 1e-9 / (jnp.exp(-theta) + 1.0)
  w16 = jnp.broadcast_to(weights, (E, NC))
  zerosD = jnp.zeros((M, D), jnp.float32)

  tbl, feat0 = _stem(x, latp, W_in, W_out, Ws1, b1r, Ws2, b2r,
                     attn_l, attn_r, s_attn)

  gs = _gatherD(tbl, src)
  gd = _gatherD(tbl, dst)
  ew, lp = _edge1(gs, gd, w16, wab, betaw)

  src_lo, src_hi = _split_idx(src.reshape(2500, 128))
  dst_lo, dst_hi = _split_idx(dst.reshape(2500, 128))
  src_lo = src_lo.reshape(E)
  src_hi = src_hi.reshape(E)
  dst_lo = dst_lo.reshape(E)
  dst_hi = dst_hi.reshape(E)

  def seg_sum_halves(vals, idx_lo, idx_hi):
    lo = _scatterD(vals, idx_lo, zerosD).reshape(2, M, D)[:, :HALF]
    hi = _scatterD(vals, idx_hi, zerosD).reshape(2, M, D)[:, :HALF]
    return lo, hi

  ow_lo, ow_hi = seg_sum_halves(ew, src_lo, src_hi)
  iw_lo, iw_hi = seg_sum_halves(ew, dst_lo, dst_hi)
  ns_lo, nd_lo = _norm(ow_lo, iw_lo)
  ns_hi, nd_hi = _norm(ow_hi, iw_hi)
  ns = jnp.concatenate([ns_lo, ns_hi], axis=0)
  nd = jnp.concatenate([nd_lo, nd_hi], axis=0)
  psrc = _gatherD(ns, src)
  pdst = _gatherD(nd, dst)
  w = _wmul(psrc, pdst, ew, perm)

  f0_lo = feat0[:HALF]
  f0_hi = feat0[HALF:]
  feat = feat0
  for _ in range(8):
    fs = _gatherD(feat, src)
    m = _mul(fs, w)
    agg_lo, agg_hi = seg_sum_halves(m, dst_lo, dst_hi)
    feat = jnp.concatenate([_axpy(agg_lo, f0_lo), _axpy(agg_hi, f0_hi)],
                           axis=0)

  out = _final(feat)
  return (out, lp[0, 0])


# speedup vs baseline: 3.2349x; 1.2596x over previous
import functools

import jax
import jax.numpy as jnp
from jax import lax
from jax.experimental import pallas as pl
from jax.experimental.pallas import tpu as pltpu
from jax.experimental.pallas import tpu_sc as plsc

N = 10000
E = 320000
NFEAT = 128
NHID = 256
NC = 16
D = 128
ALPHA = 0.1
NEG_SLOPE = 0.2

NCORE = 2
NSUB = 16
NW = NCORE * NSUB
BPW = E // NW
CH = 400

HALF = N // 2
M = 5200

BN = 1000
BE = 3200


def _make_gather():
  nchunk = BPW // CH
  mesh = plsc.VectorSubcoreMesh(core_axis_name="c", subcore_axis_name="s")

  @functools.partial(
      pl.kernel, mesh=mesh,
      out_type=jax.ShapeDtypeStruct((E, D), jnp.float32),
      scratch_types=[
          pltpu.VMEM((CH,), jnp.int32),
          pltpu.VMEM((CH, D), jnp.float32),
          pltpu.SemaphoreType.DMA,
      ],
  )
  def gk(table_hbm, idx_hbm, out_hbm, idx_v, rows_v, sem):
    wid = lax.axis_index("s") * NCORE + lax.axis_index("c")
    base = wid * BPW
    for j in range(nchunk):
      off = base + j * CH
      pltpu.sync_copy(idx_hbm.at[pl.ds(off, CH)], idx_v)
      pltpu.async_copy(table_hbm.at[idx_v], rows_v, sem).wait()
      pltpu.sync_copy(rows_v, out_hbm.at[pl.ds(off, CH)])

  return gk


def _make_scatter_add():
  nchunk = BPW // CH
  zch = 400
  nzch = M // zch
  mesh = plsc.VectorSubcoreMesh(core_axis_name="c", subcore_axis_name="s")

  @functools.partial(
      pl.kernel, mesh=mesh,
      out_type=jax.ShapeDtypeStruct((2 * M, D), jnp.float32),
      scratch_types=[
          pltpu.VMEM((CH,), jnp.int32),
          pltpu.VMEM((CH, D), jnp.float32),
          pltpu.VMEM_SHARED((M, D), jnp.float32),
      ],
  )
  def sk(vals_hbm, idx_hbm, zeros_hbm, out_hbm, idx_v, vals_v, acc_sh):
    cid = lax.axis_index("c")
    sid = lax.axis_index("s")
    wid = sid * NCORE + cid

    @pl.when(sid < nzch)
    def _():
      pltpu.sync_copy(zeros_hbm.at[pl.ds(sid * zch, zch)], vals_v)
      pltpu.sync_copy(vals_v, acc_sh.at[pl.ds(sid * zch, zch)])

    plsc.subcore_barrier()
    base = wid * BPW
    for j in range(nchunk):
      off = base + j * CH
      pltpu.sync_copy(idx_hbm.at[pl.ds(off, CH)], idx_v)
      pltpu.sync_copy(vals_hbm.at[pl.ds(off, CH)], vals_v)
      pltpu.sync_copy(vals_v, acc_sh.at[idx_v], add=True)
    plsc.subcore_barrier()

    @pl.when(sid < nzch)
    def _():
      pltpu.sync_copy(acc_sh.at[pl.ds(sid * zch, zch)], vals_v)
      pltpu.sync_copy(vals_v, out_hbm.at[pl.ds(cid * M + sid * zch, zch)])

  return sk


_gatherD = _make_gather()
_scatterD = _make_scatter_add()


def _elu(v):
  return jnp.where(v > 0, v, jnp.exp(v) - 1.0)


def _stem_kern(x_ref, latp_ref, win_ref, wout_ref, ws1_ref, b1_ref, ws2_ref,
               b2_ref, al_ref, ar_ref, sa_ref, t_ref, f0_ref):
  h1 = _elu(jnp.dot(x_ref[...], win_ref[...],
                    preferred_element_type=jnp.float32))
  h = jnp.dot(h1, wout_ref[...], preferred_element_type=jnp.float32)
  hl1 = _elu(jnp.dot(latp_ref[...], ws1_ref[...],
                     preferred_element_type=jnp.float32) + b1_ref[...])
  hl = _elu(jnp.dot(hl1, ws2_ref[...],
                    preferred_element_type=jnp.float32) + b2_ref[...])
  lr = jnp.where(h > 0, h, NEG_SLOPE * h)
  el = jnp.sum(lr * al_ref[...], axis=1, keepdims=True)
  er = jnp.sum(lr * ar_ref[...], axis=1, keepdims=True)
  ones16 = jnp.ones((1, 16), dtype=jnp.float32)
  t_ref[:, 0:16] = h
  t_ref[:, 16:32] = hl
  t_ref[:, 32:48] = hl * sa_ref[...]
  t_ref[:, 48:64] = el * ones16
  t_ref[:, 64:80] = er * ones16
  t_ref[:, 80:128] = jnp.zeros((t_ref.shape[0], 48), jnp.float32)
  f0_ref[:, 0:16] = h
  f0_ref[:, 16:128] = jnp.zeros((f0_ref.shape[0], 112), jnp.float32)


def _stem(x, latp, W_in, W_out, Ws1, b1, Ws2, b2, attn_l, attn_r, s_attn):
  full = lambda i: (0, 0)
  return pl.pallas_call(
      _stem_kern,
      grid=(N // BN,),
      in_specs=[
          pl.BlockSpec((BN, NFEAT), lambda i: (i, 0)),
          pl.BlockSpec((BN, NC), lambda i: (i, 0)),
          pl.BlockSpec((NFEAT, NHID), full),
          pl.BlockSpec((NHID, NC), full),
          pl.BlockSpec((NC, NC), full),
          pl.BlockSpec((1, NC), full),
          pl.BlockSpec((NC, NC), full),
          pl.BlockSpec((1, NC), full),
          pl.BlockSpec((1, NC), full),
          pl.BlockSpec((1, NC), full),
          pl.BlockSpec((1, NC), full),
      ],
      out_specs=[
          pl.BlockSpec((BN, D), lambda i: (i, 0)),
          pl.BlockSpec((BN, D), lambda i: (i, 0)),
      ],
      out_shape=[
          jax.ShapeDtypeStruct((N, D), jnp.float32),
          jax.ShapeDtypeStruct((N, D), jnp.float32),
      ],
  )(x, latp, W_in, W_out, Ws1, b1, Ws2, b2, attn_l, attn_r, s_attn)


def _edge1_kern(gs_ref, gd_ref, w16_ref, wab_ref, betaw_ref, ew_ref, lp_ref):
  i = pl.program_id(0)
  gs = gs_ref[...]
  gd = gd_ref[...]
  fs = gs[:, 0:16]
  fd = gd[:, 0:16]
  hls = gs[:, 16:32]
  hld = gd[:, 16:32]
  sels = gs[:, 32:48]
  se = jnp.sum(sels * hld, axis=1, keepdims=True)
  e = gs[:, 48:49] + gd[:, 64:65] + se
  sdf = jnp.sum((fs - fd) ** 2, axis=1, keepdims=True)
  sds = jnp.sum((hls - hld) ** 2, axis=1, keepdims=True)
  d = wab_ref[0:1, 0:1] * sdf + wab_ref[0:1, 1:2] * sds
  ew = jnp.exp(e - betaw_ref[0:1, 0:1] * d) + 1e-9
  ew_ref[...] = jnp.broadcast_to(ew, ew_ref.shape)

  @pl.when(i == 0)
  def _():
    lp_ref[...] = jnp.zeros_like(lp_ref)

  lp_ref[...] += jnp.sum(hls * hld * w16_ref[...])[None, None]


def _edge1(gs, gd, w16, wab, betaw):
  full = lambda i: (0, 0)
  return pl.pallas_call(
      _edge1_kern,
      grid=(E // BE,),
      in_specs=[
          pl.BlockSpec((BE, D), lambda i: (i, 0)),
          pl.BlockSpec((BE, D), lambda i: (i, 0)),
          pl.BlockSpec((BE, NC), lambda i: (i, 0)),
          pl.BlockSpec((1, 2), full),
          pl.BlockSpec((1, 1), full),
      ],
      out_specs=[
          pl.BlockSpec((BE, D), lambda i: (i, 0)),
          pl.BlockSpec((1, 1), full),
      ],
      out_shape=[
          jax.ShapeDtypeStruct((E, D), jnp.float32),
          jax.ShapeDtypeStruct((1, 1), jnp.float32),
      ],
  )(gs, gd, w16, wab, betaw)


def _norm_kern(ow_ref, iw_ref, ns_ref, nd_ref):
  ow = ow_ref[0] + ow_ref[1]
  iw = iw_ref[0] + iw_ref[1]
  ns_ref[...] = lax.rsqrt(jnp.maximum(ow, 1e-12))
  nd_ref[...] = lax.rsqrt(jnp.maximum(iw, 1e-12))


def _norm(ow2, iw2):
  return pl.pallas_call(
      _norm_kern,
      grid=(HALF // BN,),
      in_specs=[
          pl.BlockSpec((2, BN, D), lambda i: (0, i, 0)),
          pl.BlockSpec((2, BN, D), lambda i: (0, i, 0)),
      ],
      out_specs=[
          pl.BlockSpec((BN, D), lambda i: (i, 0)),
          pl.BlockSpec((BN, D), lambda i: (i, 0)),
      ],
      out_shape=[
          jax.ShapeDtypeStruct((HALF, D), jnp.float32),
          jax.ShapeDtypeStruct((HALF, D), jnp.float32),
      ],
  )(ow2, iw2)


def _wmul_kern(ps_ref, pd_ref, ew_ref, perm_ref, w_ref):
  w_ref[...] = ps_ref[...] * pd_ref[...] * ew_ref[...] + perm_ref[0:1, 0:1]


def _wmul(ps, pd, ew, perm):
  full = lambda i: (0, 0)
  return pl.pallas_call(
      _wmul_kern,
      grid=(E // BE,),
      in_specs=[
          pl.BlockSpec((BE, D), lambda i: (i, 0)),
          pl.BlockSpec((BE, D), lambda i: (i, 0)),
          pl.BlockSpec((BE, D), lambda i: (i, 0)),
          pl.BlockSpec((1, 1), full),
      ],
      out_specs=pl.BlockSpec((BE, D), lambda i: (i, 0)),
      out_shape=jax.ShapeDtypeStruct((E, D), jnp.float32),
  )(ps, pd, ew, perm)


def _mul_kern(a_ref, b_ref, o_ref):
  o_ref[...] = a_ref[...] * b_ref[...]


def _mul(a, b):
  return pl.pallas_call(
      _mul_kern,
      grid=(E // BE,),
      in_specs=[
          pl.BlockSpec((BE, D), lambda i: (i, 0)),
          pl.BlockSpec((BE, D), lambda i: (i, 0)),
      ],
      out_specs=pl.BlockSpec((BE, D), lambda i: (i, 0)),
      out_shape=jax.ShapeDtypeStruct((E, D), jnp.float32),
  )(a, b)


def _axpy_kern(agg_ref, f0_ref, o_ref):
  agg = agg_ref[0] + agg_ref[1]
  o_ref[...] = (1.0 - ALPHA) * agg + ALPHA * f0_ref[...]


def _axpy(agg2, feat0):
  return pl.pallas_call(
      _axpy_kern,
      grid=(HALF // BN,),
      in_specs=[
          pl.BlockSpec((2, BN, D), lambda i: (0, i, 0)),
          pl.BlockSpec((BN, D), lambda i: (i, 0)),
      ],
      out_specs=pl.BlockSpec((BN, D), lambda i: (i, 0)),
      out_shape=jax.ShapeDtypeStruct((HALF, D), jnp.float32),
  )(agg2, feat0)


def _split_kern(v_ref, lo_ref, hi_ref):
  v = v_ref[...]
  lo = jnp.where(v < HALF, v, HALF)
  hi = jnp.where(v >= HALF, v - HALF, HALF)
  lo_ref[...] = lo
  hi_ref[...] = hi


def _split_idx(v2d):
  return pl.pallas_call(
      _split_kern,
      grid=(1,),
      in_specs=[pl.BlockSpec((2500, 128), lambda i: (0, 0))],
      out_specs=[
          pl.BlockSpec((2500, 128), lambda i: (0, 0)),
          pl.BlockSpec((2500, 128), lambda i: (0, 0)),
      ],
      out_shape=[
          jax.ShapeDtypeStruct((2500, 128), jnp.int32),
          jax.ShapeDtypeStruct((2500, 128), jnp.int32),
      ],
  )(v2d)


def _final_kern(f_ref, o_ref):
  y = _elu(f_ref[:, 0:16])
  m = jnp.max(y, axis=1, keepdims=True)
  z = y - m
  o_ref[...] = z - jnp.log(jnp.sum(jnp.exp(z), axis=1, keepdims=True))


def _final(feat):
  return pl.pallas_call(
      _final_kern,
      grid=(N // BN,),
      in_specs=[pl.BlockSpec((BN, D), lambda i: (i, 0))],
      out_specs=pl.BlockSpec((BN, NC), lambda i: (i, 0)),
      out_shape=jax.ShapeDtypeStruct((N, NC), jnp.float32),
  )(feat)


def kernel(x, edge_index, weights, W_in, W_out, latp, Ws1, b1, Ws2, b2,
           attn_l, attn_r, s_attn, beta, aw, theta):
  src = edge_index[0]
  dst = edge_index[1]
  b1r = b1.reshape(1, NC)
  b2r = b2.reshape(1, NC)

  wab = jax.nn.softmax(aw, axis=1)
  betaw = 2.0 / (jnp.exp(-beta) + 1.0)
  perm = 1e-9 / (jnp.exp(-theta) + 1.0)
  w16 = jnp.broadcast_to(weights, (E, NC))
  zerosD = jnp.zeros((M, D), jnp.float32)

  tbl, feat0 = _stem(x, latp, W_in, W_out, Ws1, b1r, Ws2, b2r,
                     attn_l, attn_r, s_attn)

  gs = _gatherD(tbl, src)
  gd = _gatherD(tbl, dst)
  ew, lp = _edge1(gs, gd, w16, wab, betaw)

  src_lo, src_hi = _split_idx(src.reshape(2500, 128))
  dst_lo, dst_hi = _split_idx(dst.reshape(2500, 128))
  src_lo = src_lo.reshape(E)
  src_hi = src_hi.reshape(E)
  dst_lo = dst_lo.reshape(E)
  dst_hi = dst_hi.reshape(E)

  def seg_sum_halves(vals, idx_lo, idx_hi):
    lo = _scatterD(vals, idx_lo, zerosD).reshape(2, M, D)[:, :HALF]
    hi = _scatterD(vals, idx_hi, zerosD).reshape(2, M, D)[:, :HALF]
    return lo, hi

  ow_lo, ow_hi = seg_sum_halves(ew, src_lo, src_hi)
  iw_lo, iw_hi = seg_sum_halves(ew, dst_lo, dst_hi)
  ns_lo, nd_lo = _norm(ow_lo, iw_lo)
  ns_hi, nd_hi = _norm(ow_hi, iw_hi)
  ns = jnp.concatenate([ns_lo, ns_hi], axis=0)
  nd = jnp.concatenate([nd_lo, nd_hi], axis=0)
  psrc = _gatherD(ns, src)
  pdst = _gatherD(nd, dst)
  w = _wmul(psrc, pdst, ew, perm)

  f0_lo = feat0[:HALF]
  f0_hi = feat0[HALF:]
  feat = feat0
  for _ in range(8):
    fs = _gatherD(feat, src)
    m = _mul(fs, w)
    agg_lo, agg_hi = seg_sum_halves(m, dst_lo, dst_hi)
    feat = jnp.concatenate([_axpy(agg_lo, f0_lo), _axpy(agg_hi, f0_hi)],
                           axis=0)

  out = _final(feat)
  return (out, lp[0, 0])


# speedup vs baseline: 3.3271x; 1.0285x over previous
import functools

import jax
import jax.numpy as jnp
from jax import lax
from jax.experimental import pallas as pl
from jax.experimental.pallas import tpu as pltpu
from jax.experimental.pallas import tpu_sc as plsc

N = 10000
E = 320000
NFEAT = 128
NHID = 256
NC = 16
D = 128
ALPHA = 0.1
NEG_SLOPE = 0.2

NCORE = 2
NSUB = 16
NW = NCORE * NSUB
BPW = E // NW
CH = 400

HALF = N // 2
M = 5200

BN = 1000
BE = 3200


def _make_gather():
  gch = 1000
  nchunk = BPW // gch
  mesh = plsc.VectorSubcoreMesh(core_axis_name="c", subcore_axis_name="s")

  @functools.partial(
      pl.kernel, mesh=mesh,
      out_type=jax.ShapeDtypeStruct((E, D), jnp.float32),
      scratch_types=[
          pltpu.VMEM((gch,), jnp.int32),
          pltpu.VMEM((gch, D), jnp.float32),
          pltpu.SemaphoreType.DMA,
      ],
  )
  def gk(table_hbm, idx_hbm, out_hbm, idx_v, rows_v, sem):
    wid = lax.axis_index("s") * NCORE + lax.axis_index("c")
    base = wid * BPW
    for j in range(nchunk):
      off = base + j * gch
      pltpu.sync_copy(idx_hbm.at[pl.ds(off, gch)], idx_v)
      pltpu.async_copy(table_hbm.at[idx_v], rows_v, sem).wait()
      pltpu.sync_copy(rows_v, out_hbm.at[pl.ds(off, gch)])

  return gk


def _make_scatter_add():
  nchunk = BPW // CH
  zch = 400
  nzch = M // zch
  mesh = plsc.VectorSubcoreMesh(core_axis_name="c", subcore_axis_name="s")

  @functools.partial(
      pl.kernel, mesh=mesh,
      out_type=jax.ShapeDtypeStruct((2 * M, D), jnp.float32),
      scratch_types=[
          pltpu.VMEM((CH,), jnp.int32),
          pltpu.VMEM((CH, D), jnp.float32),
          pltpu.VMEM_SHARED((M, D), jnp.float32),
      ],
  )
  def sk(vals_hbm, idx_hbm, zeros_hbm, out_hbm, idx_v, vals_v, acc_sh):
    cid = lax.axis_index("c")
    sid = lax.axis_index("s")
    wid = sid * NCORE + cid

    @pl.when(sid < nzch)
    def _():
      pltpu.sync_copy(zeros_hbm.at[pl.ds(sid * zch, zch)], vals_v)
      pltpu.sync_copy(vals_v, acc_sh.at[pl.ds(sid * zch, zch)])

    plsc.subcore_barrier()
    base = wid * BPW
    for j in range(nchunk):
      off = base + j * CH
      pltpu.sync_copy(idx_hbm.at[pl.ds(off, CH)], idx_v)
      pltpu.sync_copy(vals_hbm.at[pl.ds(off, CH)], vals_v)
      pltpu.sync_copy(vals_v, acc_sh.at[idx_v], add=True)
    plsc.subcore_barrier()

    @pl.when(sid < nzch)
    def _():
      pltpu.sync_copy(acc_sh.at[pl.ds(sid * zch, zch)], vals_v)
      pltpu.sync_copy(vals_v, out_hbm.at[pl.ds(cid * M + sid * zch, zch)])

  return sk


_gatherD = _make_gather()
_scatterD = _make_scatter_add()


def _elu(v):
  return jnp.where(v > 0, v, jnp.exp(v) - 1.0)


def _stem_kern(x_ref, latp_ref, win_ref, wout_ref, ws1_ref, b1_ref, ws2_ref,
               b2_ref, al_ref, ar_ref, sa_ref, t_ref, f0_ref):
  h1 = _elu(jnp.dot(x_ref[...], win_ref[...],
                    preferred_element_type=jnp.float32))
  h = jnp.dot(h1, wout_ref[...], preferred_element_type=jnp.float32)
  hl1 = _elu(jnp.dot(latp_ref[...], ws1_ref[...],
                     preferred_element_type=jnp.float32) + b1_ref[...])
  hl = _elu(jnp.dot(hl1, ws2_ref[...],
                    preferred_element_type=jnp.float32) + b2_ref[...])
  lr = jnp.where(h > 0, h, NEG_SLOPE * h)
  el = jnp.sum(lr * al_ref[...], axis=1, keepdims=True)
  er = jnp.sum(lr * ar_ref[...], axis=1, keepdims=True)
  ones16 = jnp.ones((1, 16), dtype=jnp.float32)
  t_ref[:, 0:16] = h
  t_ref[:, 16:32] = hl
  t_ref[:, 32:48] = hl * sa_ref[...]
  t_ref[:, 48:64] = el * ones16
  t_ref[:, 64:80] = er * ones16
  t_ref[:, 80:128] = jnp.zeros((t_ref.shape[0], 48), jnp.float32)
  f0_ref[:, 0:16] = h
  f0_ref[:, 16:128] = jnp.zeros((f0_ref.shape[0], 112), jnp.float32)


def _stem(x, latp, W_in, W_out, Ws1, b1, Ws2, b2, attn_l, attn_r, s_attn):
  full = lambda i: (0, 0)
  return pl.pallas_call(
      _stem_kern,
      grid=(N // BN,),
      in_specs=[
          pl.BlockSpec((BN, NFEAT), lambda i: (i, 0)),
          pl.BlockSpec((BN, NC), lambda i: (i, 0)),
          pl.BlockSpec((NFEAT, NHID), full),
          pl.BlockSpec((NHID, NC), full),
          pl.BlockSpec((NC, NC), full),
          pl.BlockSpec((1, NC), full),
          pl.BlockSpec((NC, NC), full),
          pl.BlockSpec((1, NC), full),
          pl.BlockSpec((1, NC), full),
          pl.BlockSpec((1, NC), full),
          pl.BlockSpec((1, NC), full),
      ],
      out_specs=[
          pl.BlockSpec((BN, D), lambda i: (i, 0)),
          pl.BlockSpec((BN, D), lambda i: (i, 0)),
      ],
      out_shape=[
          jax.ShapeDtypeStruct((N, D), jnp.float32),
          jax.ShapeDtypeStruct((N, D), jnp.float32),
      ],
  )(x, latp, W_in, W_out, Ws1, b1, Ws2, b2, attn_l, attn_r, s_attn)


def _edge1_kern(gs_ref, gd_ref, w16_ref, wab_ref, betaw_ref, ew_ref, lp_ref):
  i = pl.program_id(0)
  gs = gs_ref[...]
  gd = gd_ref[...]
  fs = gs[:, 0:16]
  fd = gd[:, 0:16]
  hls = gs[:, 16:32]
  hld = gd[:, 16:32]
  sels = gs[:, 32:48]
  se = jnp.sum(sels * hld, axis=1, keepdims=True)
  e = gs[:, 48:49] + gd[:, 64:65] + se
  sdf = jnp.sum((fs - fd) ** 2, axis=1, keepdims=True)
  sds = jnp.sum((hls - hld) ** 2, axis=1, keepdims=True)
  d = wab_ref[0:1, 0:1] * sdf + wab_ref[0:1, 1:2] * sds
  ew = jnp.exp(e - betaw_ref[0:1, 0:1] * d) + 1e-9
  ew_ref[...] = jnp.broadcast_to(ew, ew_ref.shape)

  @pl.when(i == 0)
  def _():
    lp_ref[...] = jnp.zeros_like(lp_ref)

  lp_ref[...] += jnp.sum(hls * hld * w16_ref[...])[None, None]


def _edge1(gs, gd, w16, wab, betaw):
  full = lambda i: (0, 0)
  return pl.pallas_call(
      _edge1_kern,
      grid=(E // BE,),
      in_specs=[
          pl.BlockSpec((BE, D), lambda i: (i, 0)),
          pl.BlockSpec((BE, D), lambda i: (i, 0)),
          pl.BlockSpec((BE, NC), lambda i: (i, 0)),
          pl.BlockSpec((1, 2), full),
          pl.BlockSpec((1, 1), full),
      ],
      out_specs=[
          pl.BlockSpec((BE, D), lambda i: (i, 0)),
          pl.BlockSpec((1, 1), full),
      ],
      out_shape=[
          jax.ShapeDtypeStruct((E, D), jnp.float32),
          jax.ShapeDtypeStruct((1, 1), jnp.float32),
      ],
  )(gs, gd, w16, wab, betaw)


def _norm_kern(ow_ref, iw_ref, ns_ref, nd_ref):
  ow = ow_ref[0] + ow_ref[1]
  iw = iw_ref[0] + iw_ref[1]
  ns_ref[...] = lax.rsqrt(jnp.maximum(ow, 1e-12))
  nd_ref[...] = lax.rsqrt(jnp.maximum(iw, 1e-12))


def _norm(ow2, iw2):
  return pl.pallas_call(
      _norm_kern,
      grid=(HALF // BN,),
      in_specs=[
          pl.BlockSpec((2, BN, D), lambda i: (0, i, 0)),
          pl.BlockSpec((2, BN, D), lambda i: (0, i, 0)),
      ],
      out_specs=[
          pl.BlockSpec((BN, D), lambda i: (i, 0)),
          pl.BlockSpec((BN, D), lambda i: (i, 0)),
      ],
      out_shape=[
          jax.ShapeDtypeStruct((HALF, D), jnp.float32),
          jax.ShapeDtypeStruct((HALF, D), jnp.float32),
      ],
  )(ow2, iw2)


def _wmul_kern(ps_ref, pd_ref, ew_ref, perm_ref, w_ref):
  w_ref[...] = ps_ref[...] * pd_ref[...] * ew_ref[...] + perm_ref[0:1, 0:1]


def _wmul(ps, pd, ew, perm):
  full = lambda i: (0, 0)
  return pl.pallas_call(
      _wmul_kern,
      grid=(E // BE,),
      in_specs=[
          pl.BlockSpec((BE, D), lambda i: (i, 0)),
          pl.BlockSpec((BE, D), lambda i: (i, 0)),
          pl.BlockSpec((BE, D), lambda i: (i, 0)),
          pl.BlockSpec((1, 1), full),
      ],
      out_specs=pl.BlockSpec((BE, D), lambda i: (i, 0)),
      out_shape=jax.ShapeDtypeStruct((E, D), jnp.float32),
  )(ps, pd, ew, perm)


def _mul_kern(a_ref, b_ref, o_ref):
  o_ref[...] = a_ref[...] * b_ref[...]


def _mul(a, b):
  return pl.pallas_call(
      _mul_kern,
      grid=(E // BE,),
      in_specs=[
          pl.BlockSpec((BE, D), lambda i: (i, 0)),
          pl.BlockSpec((BE, D), lambda i: (i, 0)),
      ],
      out_specs=pl.BlockSpec((BE, D), lambda i: (i, 0)),
      out_shape=jax.ShapeDtypeStruct((E, D), jnp.float32),
  )(a, b)


def _axpy_kern(agg_ref, f0_ref, o_ref):
  agg = agg_ref[0] + agg_ref[1]
  o_ref[...] = (1.0 - ALPHA) * agg + ALPHA * f0_ref[...]


def _axpy(agg2, feat0):
  return pl.pallas_call(
      _axpy_kern,
      grid=(HALF // BN,),
      in_specs=[
          pl.BlockSpec((2, BN, D), lambda i: (0, i, 0)),
          pl.BlockSpec((BN, D), lambda i: (i, 0)),
      ],
      out_specs=pl.BlockSpec((BN, D), lambda i: (i, 0)),
      out_shape=jax.ShapeDtypeStruct((HALF, D), jnp.float32),
  )(agg2, feat0)


def _split_kern(v_ref, lo_ref, hi_ref):
  v = v_ref[...]
  lo = jnp.where(v < HALF, v, HALF)
  hi = jnp.where(v >= HALF, v - HALF, HALF)
  lo_ref[...] = lo
  hi_ref[...] = hi


def _split_idx(v2d):
  return pl.pallas_call(
      _split_kern,
      grid=(1,),
      in_specs=[pl.BlockSpec((2500, 128), lambda i: (0, 0))],
      out_specs=[
          pl.BlockSpec((2500, 128), lambda i: (0, 0)),
          pl.BlockSpec((2500, 128), lambda i: (0, 0)),
      ],
      out_shape=[
          jax.ShapeDtypeStruct((2500, 128), jnp.int32),
          jax.ShapeDtypeStruct((2500, 128), jnp.int32),
      ],
  )(v2d)


def _final_kern(f_ref, o_ref):
  y = _elu(f_ref[:, 0:16])
  m = jnp.max(y, axis=1, keepdims=True)
  z = y - m
  o_ref[...] = z - jnp.log(jnp.sum(jnp.exp(z), axis=1, keepdims=True))


def _final(feat):
  return pl.pallas_call(
      _final_kern,
      grid=(N // BN,),
      in_specs=[pl.BlockSpec((BN, D), lambda i: (i, 0))],
      out_specs=pl.BlockSpec((BN, NC), lambda i: (i, 0)),
      out_shape=jax.ShapeDtypeStruct((N, NC), jnp.float32),
  )(feat)


def kernel(x, edge_index, weights, W_in, W_out, latp, Ws1, b1, Ws2, b2,
           attn_l, attn_r, s_attn, beta, aw, theta):
  src = edge_index[0]
  dst = edge_index[1]
  b1r = b1.reshape(1, NC)
  b2r = b2.reshape(1, NC)

  wab = jax.nn.softmax(aw, axis=1)
  betaw = 2.0 / (jnp.exp(-beta) + 1.0)
  perm = 1e-9 / (jnp.exp(-theta) + 1.0)
  w16 = jnp.broadcast_to(weights, (E, NC))
  zerosD = jnp.zeros((M, D), jnp.float32)

  tbl, feat0 = _stem(x, latp, W_in, W_out, Ws1, b1r, Ws2, b2r,
                     attn_l, attn_r, s_attn)

  gs = _gatherD(tbl, src)
  gd = _gatherD(tbl, dst)
  ew, lp = _edge1(gs, gd, w16, wab, betaw)

  src_lo, src_hi = _split_idx(src.reshape(2500, 128))
  dst_lo, dst_hi = _split_idx(dst.reshape(2500, 128))
  src_lo = src_lo.reshape(E)
  src_hi = src_hi.reshape(E)
  dst_lo = dst_lo.reshape(E)
  dst_hi = dst_hi.reshape(E)

  def seg_sum_halves(vals, idx_lo, idx_hi):
    lo = _scatterD(vals, idx_lo, zerosD).reshape(2, M, D)[:, :HALF]
    hi = _scatterD(vals, idx_hi, zerosD).reshape(2, M, D)[:, :HALF]
    return lo, hi

  ow_lo, ow_hi = seg_sum_halves(ew, src_lo, src_hi)
  iw_lo, iw_hi = seg_sum_halves(ew, dst_lo, dst_hi)
  ns_lo, nd_lo = _norm(ow_lo, iw_lo)
  ns_hi, nd_hi = _norm(ow_hi, iw_hi)
  ns = jnp.concatenate([ns_lo, ns_hi], axis=0)
  nd = jnp.concatenate([nd_lo, nd_hi], axis=0)
  psrc = _gatherD(ns, src)
  pdst = _gatherD(nd, dst)
  w = _wmul(psrc, pdst, ew, perm)

  f0_lo = feat0[:HALF]
  f0_hi = feat0[HALF:]
  feat = feat0
  for _ in range(8):
    fs = _gatherD(feat, src)
    m = _mul(fs, w)
    agg_lo, agg_hi = seg_sum_halves(m, dst_lo, dst_hi)
    feat = jnp.concatenate([_axpy(agg_lo, f0_lo), _axpy(agg_hi, f0_hi)],
                           axis=0)

  out = _final(feat)
  return (out, lp[0, 0])
